# Initial kernel scaffold; baseline (speedup 1.0000x reference)
import jax
import jax.numpy as jnp
from jax.experimental import pallas as pl


def kernel(x, W_enc, b_enc, W_dec, b_dec):
    raise NotImplementedError("write your pallas kernel here")


# speedup vs baseline: 36.4099x; 36.4099x over previous
import functools

import jax
import jax.numpy as jnp
from jax import lax
from jax.experimental import pallas as pl
from jax.experimental.pallas import tpu as pltpu
from jax.experimental.pallas import tpu_sc as plsc

ACT = 768
FDIM = 16384
BATCH = 4096
TOPK = 64
KB = TOPK * BATCH
N = BATCH * FDIM

NB_COARSE = 1 << 16
NB_FINE = 1 << 15

_L = 16
_NC, _NS = 2, 16
_NW = _NC * _NS
PER_W = N // _NW
CHUNK = 16384
NCHUNK = PER_W // CHUNK

BN = 512


def _encode_body(x_ref, w_ref, benc_ref, bdec_ref, out_ref):
    xb = x_ref[...] - bdec_ref[...]
    acts = lax.dot_general(xb, w_ref[...], (((1,), (1,)), ((), ())),
                           preferred_element_type=jnp.float32)
    out_ref[...] = jnp.maximum(acts + benc_ref[...], 0.0)


def _encode(x, w_enc, b_enc2, b_dec2):
    return pl.pallas_call(
        _encode_body,
        grid=(FDIM // BN,),
        in_specs=[
            pl.BlockSpec((BATCH, ACT), lambda j: (0, 0)),
            pl.BlockSpec((BN, ACT), lambda j: (j, 0)),
            pl.BlockSpec((1, BN), lambda j: (0, j)),
            pl.BlockSpec((1, ACT), lambda j: (0, 0)),
        ],
        out_specs=pl.BlockSpec((BATCH, BN), lambda j: (0, j)),
        out_shape=jax.ShapeDtypeStruct((BATCH, FDIM), jnp.float32),
    )(x, w_enc, b_enc2, b_dec2)


def _hist_body(fine, *refs):
    if fine:
        flat_hbm, bsel_hbm, hist_hbm, buf, hist_v, bsel_v = refs
        nbins = NB_FINE
    else:
        flat_hbm, hist_hbm, buf, hist_v = refs
        nbins = NB_COARSE
    wid = lax.axis_index("s") * _NC + lax.axis_index("c")
    base = wid * PER_W

    zeros16 = jnp.zeros((_L,), jnp.int32)

    def _zero(i, c):
        hist_v[pl.ds(i * _L, _L)] = zeros16
        return c
    lax.fori_loop(0, nbins // _L, _zero, 0)

    if fine:
        pltpu.sync_copy(bsel_hbm, bsel_v)
        bsel = bsel_v[...]

    ones16 = jnp.ones((_L,), jnp.int32)

    def _chunk(c, carry):
        pltpu.sync_copy(flat_hbm.at[pl.ds(base + c * CHUNK, CHUNK)], buf)

        def _vec(k, cc):
            v = buf[pl.ds(k * _L, _L)]
            bits = lax.bitcast_convert_type(v, jnp.int32)
            coarse = lax.shift_right_logical(bits, 15)
            if fine:
                idx = bits & jnp.int32(0x7FFF)
                m = coarse == bsel
            else:
                idx = coarse
                m = bits != 0
            plsc.addupdate_scatter(hist_v, [idx], ones16, mask=m)
            return cc
        lax.fori_loop(0, CHUNK // _L, _vec, carry)
        return carry
    lax.fori_loop(0, NCHUNK, _chunk, 0)

    pltpu.sync_copy(hist_v, hist_hbm.at[wid])


def _make_hist_kernel(fine):
    nbins = NB_FINE if fine else NB_COARSE
    scratch = [
        pltpu.VMEM((CHUNK,), jnp.float32),
        pltpu.VMEM((nbins,), jnp.int32),
    ]
    if fine:
        scratch.append(pltpu.VMEM((_L,), jnp.int32))
    return pl.kernel(
        functools.partial(_hist_body, fine),
        out_type=jax.ShapeDtypeStruct((_NW, nbins), jnp.int32),
        mesh=plsc.VectorSubcoreMesh(core_axis_name="c", subcore_axis_name="s"),
        scratch_types=scratch,
        compiler_params=pltpu.CompilerParams(needs_layout_passes=False),
    )


_coarse_hist = _make_hist_kernel(False)
_fine_hist = _make_hist_kernel(True)


def _suffix_pick(hist, target):
    nb = hist.shape[0]
    suffix = jnp.cumsum(hist[::-1])[::-1]
    suffix_pad = jnp.concatenate([suffix, jnp.zeros((1,), suffix.dtype)])
    ok = suffix >= target
    b = jnp.max(jnp.where(ok, jnp.arange(nb, dtype=jnp.int32), 0))
    return b, suffix_pad[b + 1], suffix[0]


def _decode_body(tau_ref, p_ref, w_ref, bdec_ref, out_ref):
    j = pl.program_id(0)
    p = p_ref[...]
    bits = lax.bitcast_convert_type(p, jnp.int32)
    sel = jnp.where(bits >= tau_ref[0, 0], p, 0.0)
    acc = lax.dot_general(sel, w_ref[...], (((1,), (1,)), ((), ())),
                          preferred_element_type=jnp.float32)

    @pl.when(j == 0)
    def _init():
        out_ref[...] = acc + bdec_ref[...]

    @pl.when(j > 0)
    def _acc():
        out_ref[...] += acc


def _decode(tau_bits, post, w_dec, b_dec2):
    return pl.pallas_call(
        _decode_body,
        grid=(FDIM // BN,),
        in_specs=[
            pl.BlockSpec(memory_space=pltpu.SMEM),
            pl.BlockSpec((BATCH, BN), lambda j: (0, j)),
            pl.BlockSpec((ACT, BN), lambda j: (0, j)),
            pl.BlockSpec((1, ACT), lambda j: (0, 0)),
        ],
        out_specs=pl.BlockSpec((BATCH, ACT), lambda j: (0, 0)),
        out_shape=jax.ShapeDtypeStruct((BATCH, ACT), jnp.float32),
    )(tau_bits, post, w_dec, b_dec2)


def kernel(x, W_enc, b_enc, W_dec, b_dec):
    b_enc2 = b_enc.reshape(1, FDIM)
    b_dec2 = b_dec.reshape(1, ACT)

    post = _encode(x, W_enc, b_enc2, b_dec2)
    flat = post.reshape(N)

    coarse = _coarse_hist(flat).sum(axis=0)
    bstar, above, total_pos = _suffix_pick(coarse, KB)

    fineh = _fine_hist(flat, jnp.full((_L,), bstar, jnp.int32)).sum(axis=0)
    fstar, _, _ = _suffix_pick(fineh, KB - above)

    tau_bits = jnp.where(total_pos <= KB,
                         jnp.int32(1), (bstar << 15) | fstar)

    return _decode(tau_bits.reshape(1, 1), post, W_dec, b_dec2)


# speedup vs baseline: 48.8614x; 1.3420x over previous
import functools

import jax
import jax.numpy as jnp
from jax import lax
from jax.experimental import pallas as pl
from jax.experimental.pallas import tpu as pltpu
from jax.experimental.pallas import tpu_sc as plsc

ACT = 768
FDIM = 16384
BATCH = 4096
TOPK = 64
KB = TOPK * BATCH
N = BATCH * FDIM

NB_COARSE = 1 << 16
NB_FINE = 1 << 15

_L = 16
_NC, _NS = 2, 16
_NW = _NC * _NS
PER_W = N // _NW
CHUNK = 16384
NCHUNK = PER_W // CHUNK

BN = 512


def _encode_body(x_ref, w_ref, benc_ref, bdec_ref, out_ref):
    xb = x_ref[...] - bdec_ref[...]
    acts = lax.dot_general(xb, w_ref[...], (((1,), (1,)), ((), ())),
                           preferred_element_type=jnp.float32)
    out_ref[...] = jnp.maximum(acts + benc_ref[...], 0.0)


def _encode(x, w_enc, b_enc2, b_dec2):
    return pl.pallas_call(
        _encode_body,
        grid=(FDIM // BN,),
        in_specs=[
            pl.BlockSpec((BATCH, ACT), lambda j: (0, 0)),
            pl.BlockSpec((BN, ACT), lambda j: (j, 0)),
            pl.BlockSpec((1, BN), lambda j: (0, j)),
            pl.BlockSpec((1, ACT), lambda j: (0, 0)),
        ],
        out_specs=pl.BlockSpec((BATCH, BN), lambda j: (0, j)),
        out_shape=jax.ShapeDtypeStruct((BATCH, FDIM), jnp.float32),
    )(x, w_enc, b_enc2, b_dec2)


_UNROLL = 16
_ROWS_W = BATCH // _NW


def _hist_body(fine, *refs):
    if fine:
        post_hbm, bsel_hbm, hist_hbm, buf0, buf1, hist_v, bsel_v, sem0, sem1 = refs
        nbins = NB_FINE
    else:
        post_hbm, hist_hbm, buf0, buf1, hist_v, sem0, sem1 = refs
        nbins = NB_COARSE
    wid = lax.axis_index("s") * _NC + lax.axis_index("c")
    row0 = wid * _ROWS_W

    zeros16 = jnp.zeros((_L,), jnp.int32)

    def _zero(i, c):
        b = i * (_L * 8)
        for u in range(8):
            hist_v[pl.ds(b + u * _L, _L)] = zeros16
        return c
    lax.fori_loop(0, nbins // (_L * 8), _zero, 0)

    if fine:
        pltpu.sync_copy(bsel_hbm, bsel_v)
        bsel = bsel_v[...]

    ones16 = jnp.ones((_L,), jnp.int32)

    def _proc(buf):
        def _blk(k, cc):
            b = k * (_L * _UNROLL)
            for u in range(_UNROLL):
                v = buf[pl.ds(b + u * _L, _L)]
                bits = lax.bitcast_convert_type(v, jnp.int32)
                coarse = lax.shift_right_logical(bits, 15)
                if fine:
                    idx = bits & jnp.int32(0x7FFF)
                    m = coarse == bsel
                else:
                    idx = coarse
                    m = bits != 0
                plsc.addupdate_scatter(hist_v, [idx], ones16, mask=m)
            return cc
        lax.fori_loop(0, FDIM // (_L * _UNROLL), _blk, 0)

    pltpu.async_copy(post_hbm.at[row0], buf0, sem0)
    pltpu.async_copy(post_hbm.at[row0 + 1], buf1, sem1)

    def _outer(p, c):
        r = row0 + 2 * p
        pltpu.make_async_copy(post_hbm.at[0], buf0, sem0).wait()
        _proc(buf0)

        @pl.when(p < _ROWS_W // 2 - 1)
        def _s0():
            pltpu.async_copy(post_hbm.at[r + 2], buf0, sem0)

        pltpu.make_async_copy(post_hbm.at[0], buf1, sem1).wait()
        _proc(buf1)

        @pl.when(p < _ROWS_W // 2 - 1)
        def _s1():
            pltpu.async_copy(post_hbm.at[r + 3], buf1, sem1)
        return c
    lax.fori_loop(0, _ROWS_W // 2, _outer, 0)

    pltpu.sync_copy(hist_v, hist_hbm.at[wid])


def _make_hist_kernel(fine):
    nbins = NB_FINE if fine else NB_COARSE
    scratch = [
        pltpu.VMEM((FDIM,), jnp.float32),
        pltpu.VMEM((FDIM,), jnp.float32),
        pltpu.VMEM((nbins,), jnp.int32),
    ]
    if fine:
        scratch.append(pltpu.VMEM((_L,), jnp.int32))
    scratch += [pltpu.SemaphoreType.DMA, pltpu.SemaphoreType.DMA]
    return pl.kernel(
        functools.partial(_hist_body, fine),
        out_type=jax.ShapeDtypeStruct((_NW, nbins), jnp.int32),
        mesh=plsc.VectorSubcoreMesh(core_axis_name="c", subcore_axis_name="s"),
        scratch_types=scratch,
        compiler_params=pltpu.CompilerParams(needs_layout_passes=False),
    )


_coarse_hist = _make_hist_kernel(False)
_fine_hist = _make_hist_kernel(True)


def _suffix_pick(hist, target):
    nb = hist.shape[0]
    suffix = jnp.cumsum(hist[::-1])[::-1]
    suffix_pad = jnp.concatenate([suffix, jnp.zeros((1,), suffix.dtype)])
    ok = suffix >= target
    b = jnp.max(jnp.where(ok, jnp.arange(nb, dtype=jnp.int32), 0))
    return b, suffix_pad[b + 1], suffix[0]


def _decode_body(tau_ref, p_ref, w_ref, bdec_ref, out_ref):
    j = pl.program_id(0)
    p = p_ref[...]
    bits = lax.bitcast_convert_type(p, jnp.int32)
    sel = jnp.where(bits >= tau_ref[0, 0], p, 0.0)
    acc = lax.dot_general(sel, w_ref[...], (((1,), (1,)), ((), ())),
                          preferred_element_type=jnp.float32)

    @pl.when(j == 0)
    def _init():
        out_ref[...] = acc + bdec_ref[...]

    @pl.when(j > 0)
    def _acc():
        out_ref[...] += acc


def _decode(tau_bits, post, w_dec, b_dec2):
    return pl.pallas_call(
        _decode_body,
        grid=(FDIM // BN,),
        in_specs=[
            pl.BlockSpec(memory_space=pltpu.SMEM),
            pl.BlockSpec((BATCH, BN), lambda j: (0, j)),
            pl.BlockSpec((ACT, BN), lambda j: (0, j)),
            pl.BlockSpec((1, ACT), lambda j: (0, 0)),
        ],
        out_specs=pl.BlockSpec((BATCH, ACT), lambda j: (0, 0)),
        out_shape=jax.ShapeDtypeStruct((BATCH, ACT), jnp.float32),
    )(tau_bits, post, w_dec, b_dec2)


def kernel(x, W_enc, b_enc, W_dec, b_dec):
    b_enc2 = b_enc.reshape(1, FDIM)
    b_dec2 = b_dec.reshape(1, ACT)

    post = _encode(x, W_enc, b_enc2, b_dec2)

    coarse = _coarse_hist(post).sum(axis=0)
    bstar, above, total_pos = _suffix_pick(coarse, KB)

    fineh = _fine_hist(post, jnp.full((_L,), bstar, jnp.int32)).sum(axis=0)
    fstar, _, _ = _suffix_pick(fineh, KB - above)

    tau_bits = jnp.where(total_pos <= KB,
                         jnp.int32(1), (bstar << 15) | fstar)

    return _decode(tau_bits.reshape(1, 1), post, W_dec, b_dec2)


# speedup vs baseline: 167.8744x; 3.4357x over previous
import functools

import jax
import jax.numpy as jnp
from jax import lax
from jax.experimental import pallas as pl
from jax.experimental.pallas import tpu as pltpu
from jax.experimental.pallas import tpu_sc as plsc

ACT = 768
FDIM = 16384
BATCH = 4096
TOPK = 64
KB = TOPK * BATCH
N = BATCH * FDIM

NB_COARSE = 1 << 16
NB_FINE = 1 << 15

_L = 16
_NC, _NS = 2, 16
_NW = _NC * _NS
PER_W = N // _NW
CHUNK = 16384
NCHUNK = PER_W // CHUNK

BN = 512


def _encode_body(x_ref, w_ref, benc_ref, bdec_ref, out_ref):
    xb = x_ref[...] - bdec_ref[...]
    acts = lax.dot_general(xb, w_ref[...], (((1,), (1,)), ((), ())),
                           preferred_element_type=jnp.float32)
    out_ref[...] = jnp.maximum(acts + benc_ref[...], 0.0)


def _encode(x, w_enc, b_enc2, b_dec2):
    return pl.pallas_call(
        _encode_body,
        grid=(FDIM // BN,),
        in_specs=[
            pl.BlockSpec((BATCH, ACT), lambda j: (0, 0)),
            pl.BlockSpec((BN, ACT), lambda j: (j, 0)),
            pl.BlockSpec((1, BN), lambda j: (0, j)),
            pl.BlockSpec((1, ACT), lambda j: (0, 0)),
        ],
        out_specs=pl.BlockSpec((BATCH, BN), lambda j: (0, j)),
        out_shape=jax.ShapeDtypeStruct((BATCH, FDIM), jnp.float32),
    )(x, w_enc, b_enc2, b_dec2)


_UNROLL = 16
_ROWS_W = BATCH // _NW


def _hist_body(fine, *refs):
    if fine:
        post_hbm, bsel_hbm, hist_hbm, buf0, buf1, hist_v, bsel_v, sem0, sem1 = refs
        nbins = NB_FINE
    else:
        post_hbm, hist_hbm, buf0, buf1, hist_v, sem0, sem1 = refs
        nbins = NB_COARSE
    wid = lax.axis_index("s") * _NC + lax.axis_index("c")
    row0 = wid * _ROWS_W

    zeros16 = jnp.zeros((_L,), jnp.int32)

    @plsc.parallel_loop(0, nbins // _L, 1, unroll=8)
    def _zero(i):
        hist_v[pl.ds(i * _L, _L)] = zeros16

    if fine:
        pltpu.sync_copy(bsel_hbm, bsel_v)
        bsel = bsel_v[...]

    ones16 = jnp.ones((_L,), jnp.int32)

    def _proc(buf):
        @plsc.parallel_loop(0, FDIM // _L, 1, unroll=_UNROLL)
        def _vec(k):
            v = buf[pl.ds(k * _L, _L)]
            bits = lax.bitcast_convert_type(v, jnp.int32)
            coarse = lax.shift_right_logical(bits, 15)
            if fine:
                idx = bits & jnp.int32(0x7FFF)
                m = coarse == bsel
            else:
                idx = coarse
                m = bits != 0
            plsc.addupdate_scatter(hist_v, [idx], ones16, mask=m)

    pltpu.async_copy(post_hbm.at[row0], buf0, sem0)
    pltpu.async_copy(post_hbm.at[row0 + 1], buf1, sem1)

    def _outer(p, c):
        r = row0 + 2 * p
        pltpu.make_async_copy(post_hbm.at[0], buf0, sem0).wait()
        _proc(buf0)

        @pl.when(p < _ROWS_W // 2 - 1)
        def _s0():
            pltpu.async_copy(post_hbm.at[r + 2], buf0, sem0)

        pltpu.make_async_copy(post_hbm.at[0], buf1, sem1).wait()
        _proc(buf1)

        @pl.when(p < _ROWS_W // 2 - 1)
        def _s1():
            pltpu.async_copy(post_hbm.at[r + 3], buf1, sem1)
        return c
    lax.fori_loop(0, _ROWS_W // 2, _outer, 0)

    pltpu.sync_copy(hist_v, hist_hbm.at[wid])


def _make_hist_kernel(fine):
    nbins = NB_FINE if fine else NB_COARSE
    scratch = [
        pltpu.VMEM((FDIM,), jnp.float32),
        pltpu.VMEM((FDIM,), jnp.float32),
        pltpu.VMEM((nbins,), jnp.int32),
    ]
    if fine:
        scratch.append(pltpu.VMEM((_L,), jnp.int32))
    scratch += [pltpu.SemaphoreType.DMA, pltpu.SemaphoreType.DMA]
    return pl.kernel(
        functools.partial(_hist_body, fine),
        out_type=jax.ShapeDtypeStruct((_NW, nbins), jnp.int32),
        mesh=plsc.VectorSubcoreMesh(core_axis_name="c", subcore_axis_name="s"),
        scratch_types=scratch,
        compiler_params=pltpu.CompilerParams(needs_layout_passes=False),
    )


_coarse_hist = _make_hist_kernel(False)
_fine_hist = _make_hist_kernel(True)


def _suffix_pick(hist, target):
    nb = hist.shape[0]
    suffix = jnp.cumsum(hist[::-1])[::-1]
    suffix_pad = jnp.concatenate([suffix, jnp.zeros((1,), suffix.dtype)])
    ok = suffix >= target
    b = jnp.max(jnp.where(ok, jnp.arange(nb, dtype=jnp.int32), 0))
    return b, suffix_pad[b + 1], suffix[0]


def _decode_body(tau_ref, p_ref, w_ref, bdec_ref, out_ref):
    j = pl.program_id(0)
    p = p_ref[...]
    bits = lax.bitcast_convert_type(p, jnp.int32)
    sel = jnp.where(bits >= tau_ref[0, 0], p, 0.0)
    acc = lax.dot_general(sel, w_ref[...], (((1,), (1,)), ((), ())),
                          preferred_element_type=jnp.float32)

    @pl.when(j == 0)
    def _init():
        out_ref[...] = acc + bdec_ref[...]

    @pl.when(j > 0)
    def _acc():
        out_ref[...] += acc


def _decode(tau_bits, post, w_dec, b_dec2):
    return pl.pallas_call(
        _decode_body,
        grid=(FDIM // BN,),
        in_specs=[
            pl.BlockSpec(memory_space=pltpu.SMEM),
            pl.BlockSpec((BATCH, BN), lambda j: (0, j)),
            pl.BlockSpec((ACT, BN), lambda j: (0, j)),
            pl.BlockSpec((1, ACT), lambda j: (0, 0)),
        ],
        out_specs=pl.BlockSpec((BATCH, ACT), lambda j: (0, 0)),
        out_shape=jax.ShapeDtypeStruct((BATCH, ACT), jnp.float32),
    )(tau_bits, post, w_dec, b_dec2)


def kernel(x, W_enc, b_enc, W_dec, b_dec):
    b_enc2 = b_enc.reshape(1, FDIM)
    b_dec2 = b_dec.reshape(1, ACT)

    post = _encode(x, W_enc, b_enc2, b_dec2)

    coarse = _coarse_hist(post).sum(axis=0)
    bstar, above, total_pos = _suffix_pick(coarse, KB)

    fineh = _fine_hist(post, jnp.full((_L,), bstar, jnp.int32)).sum(axis=0)
    fstar, _, _ = _suffix_pick(fineh, KB - above)

    tau_bits = jnp.where(total_pos <= KB,
                         jnp.int32(1), (bstar << 15) | fstar)

    return _decode(tau_bits.reshape(1, 1), post, W_dec, b_dec2)
